# Initial kernel scaffold; baseline (speedup 1.0000x reference)
import jax
import jax.numpy as jnp
from jax.experimental import pallas as pl


def kernel(syndrome, parity_matrix, channel_llrs, w_cv, w_vc, damping):
    raise NotImplementedError("write your pallas kernel here")


# speedup vs baseline: 1.2576x; 1.2576x over previous
import functools

import jax
import jax.numpy as jnp
from jax.experimental import pallas as pl
from jax.experimental.pallas import tpu as pltpu

_B = 64
_V = 4096
_C = 4096
_ITERS = 15
_TILE = 512
_KT = _C // _TILE

_DIMS_NN = (((1,), (0,)), ((), ()))


def _gather(acc_ref):
    a = acc_ref[...]
    return jnp.concatenate([a[t] for t in range(_KT)], axis=1)


def _bp_body(syn_ref, ht_ref, h_ref, llr_ref, wcv_ref, wvc_ref, damp_ref,
             out_ref, x_ref, xb_ref, cb_ref, acc1_ref, acc2_ref):
    i = pl.program_id(0)
    p = pl.program_id(1)
    t = pl.program_id(2)
    wcv = wcv_ref[0, 0]
    wvc = wvc_ref[0, 0]
    damp = damp_ref[0, 0]

    @pl.when((p == 0) & (t == 0))
    def _():
        llr = llr_ref[...]
        ctv = wcv * _gather(acc2_ref)
        x_new = damp * x_ref[...] + (1.0 - damp) * (llr + ctv)
        x = jnp.where(i == 0, llr, x_new)
        x_ref[...] = x
        xb_ref[...] = x.astype(jnp.bfloat16)

    @pl.when(p == 0)
    def _():
        acc1_ref[t] = jax.lax.dot_general(
            xb_ref[...], ht_ref[...], _DIMS_NN,
            preferred_element_type=jnp.float32)

    @pl.when((p == 1) & (t == 0))
    def _():
        v_to_c = wvc * _gather(acc1_ref)
        s_sign = 1.0 - 2.0 * syn_ref[...].astype(jnp.float32)
        c_msg = s_sign * jnp.tanh(v_to_c * 0.5)
        cb_ref[...] = c_msg.astype(jnp.bfloat16)

    @pl.when(p == 1)
    def _():
        acc2_ref[t] = jax.lax.dot_general(
            cb_ref[...], h_ref[...], _DIMS_NN,
            preferred_element_type=jnp.float32)

    @pl.when((i == _ITERS - 1) & (p == 1) & (t == _KT - 1))
    def _():
        llr = llr_ref[...]
        ctv = wcv * _gather(acc2_ref)
        x_fin = damp * x_ref[...] + (1.0 - damp) * (llr + ctv)
        out_ref[...] = jax.nn.sigmoid(-x_fin)


@functools.partial(jax.jit, static_argnames=())
def kernel(syndrome, parity_matrix, channel_llrs, w_cv, w_vc, damping):
    h_bf = parity_matrix.astype(jnp.bfloat16)
    ht_bf = h_bf.T
    wcv = jnp.reshape(w_cv.astype(jnp.float32), (1, 1))
    wvc = jnp.reshape(w_vc.astype(jnp.float32), (1, 1))
    damp = jnp.reshape(damping.astype(jnp.float32), (1, 1))
    out = pl.pallas_call(
        _bp_body,
        grid=(_ITERS, 2, _KT),
        out_shape=jax.ShapeDtypeStruct((_B, _V), jnp.float32),
        in_specs=[
            pl.BlockSpec((_B, _C), lambda i, p, t: (0, 0)),
            pl.BlockSpec((_V, _TILE), lambda i, p, t: (0, jnp.where(p == 0, t, _KT - 1))),
            pl.BlockSpec((_C, _TILE), lambda i, p, t: (0, jnp.where(p == 1, t, 0))),
            pl.BlockSpec((_B, _V), lambda i, p, t: (0, 0)),
            pl.BlockSpec((1, 1), lambda i, p, t: (0, 0), memory_space=pltpu.SMEM),
            pl.BlockSpec((1, 1), lambda i, p, t: (0, 0), memory_space=pltpu.SMEM),
            pl.BlockSpec((1, 1), lambda i, p, t: (0, 0), memory_space=pltpu.SMEM),
        ],
        out_specs=pl.BlockSpec((_B, _V), lambda i, p, t: (0, 0)),
        scratch_shapes=[
            pltpu.VMEM((_B, _V), jnp.float32),
            pltpu.VMEM((_B, _V), jnp.bfloat16),
            pltpu.VMEM((_B, _C), jnp.bfloat16),
            pltpu.VMEM((_KT, _B, _TILE), jnp.float32),
            pltpu.VMEM((_KT, _B, _TILE), jnp.float32),
        ],
    )(syndrome, ht_bf, h_bf, channel_llrs, wcv, wvc, damp)
    return out
